# baseline (speedup 1.0000x reference)
import functools

import jax
import jax.numpy as jnp
from jax import lax
from jax.experimental import pallas as pl
from jax.experimental.pallas import tpu as pltpu
from jax.experimental.pallas import tpu_sc as plsc

_NC = 2
_NS = 16
_NW = _NC * _NS
_C = 128


_RPT = 80
_PAD = 8


def _sc_degree(dst2, zeros_np):
    R, C = dst2.shape
    (Np,) = zeros_np.shape
    N = Np - _PAD

    mesh = plsc.VectorSubcoreMesh(core_axis_name="c", subcore_axis_name="s")

    @functools.partial(
        pl.kernel,
        out_type=(
            jax.ShapeDtypeStruct((Np,), jnp.float32),
            jax.ShapeDtypeStruct((Np,), jnp.float32),
        ),
        mesh=mesh,
        scratch_types=[
            pltpu.VMEM((_RPT, C), jnp.int32),
            pltpu.VMEM((C,), jnp.float32),
            pltpu.VMEM_SHARED((Np,), jnp.float32),
            pltpu.SemaphoreType.DMA,
            pltpu.SemaphoreType.DMA,
        ],
    )
    def k(dst_hbm, z_hbm, out0_hbm, out1_hbm, didx, ones, acc, s0, s1):
        c = lax.axis_index("c")
        s = lax.axis_index("s")
        wid = s * _NC + c
        row0 = pl.multiple_of(wid * _RPT, 8)
        ss = [s0, s1]
        for i in range(C // 16):
            ones[pl.ds(i * 16, 16)] = jnp.ones((16,), jnp.float32)

        pltpu.sync_copy(dst_hbm.at[pl.ds(row0, _RPT)], didx)

        @pl.when(s == 0)
        def _():
            pltpu.sync_copy(z_hbm, acc)

        plsc.subcore_barrier()

        def step(j, b, o):
            @pl.when(j > 0)
            def _():
                pltpu.make_async_copy(ones, acc.at[didx.at[j - 1]], ss[o]).wait()

            pltpu.async_copy(ones, acc.at[didx.at[j]], ss[b], add=True)

        def body(i, carry):
            step(2 * i, 0, 1)
            step(2 * i + 1, 1, 0)
            return carry

        lax.fori_loop(0, _RPT // 2, body, 0)
        pltpu.make_async_copy(ones, acc.at[didx.at[_RPT - 1]], ss[1]).wait()

        plsc.subcore_barrier()

        @pl.when(jnp.logical_and(s == 0, c == 0))
        def _():
            pltpu.sync_copy(acc, out0_hbm)

        @pl.when(jnp.logical_and(s == 0, c == 1))
        def _():
            pltpu.sync_copy(acc, out1_hbm)

    return k(dst2, zeros_np)


def _sc_aggregate(y, srcflat, dst2, zeros_npd):
    N, D = y.shape
    R, C = dst2.shape
    Np = N + _PAD
    rpt8 = (N // (_NS * 8)) * 8
    rem = N - _NS * rpt8

    mesh = plsc.VectorSubcoreMesh(core_axis_name="c", subcore_axis_name="s")

    @functools.partial(
        pl.kernel,
        out_type=jax.ShapeDtypeStruct((_NC, N, D), jnp.float32),
        mesh=mesh,
        scratch_types=[
            pltpu.VMEM((C,), jnp.int32),
            pltpu.VMEM((C,), jnp.int32),
            pltpu.VMEM((_RPT, C), jnp.int32),
            pltpu.VMEM((C, D), jnp.float32),
            pltpu.VMEM((C, D), jnp.float32),
            pltpu.VMEM_SHARED((Np, D), jnp.float32),
            pltpu.SemaphoreType.DMA,
            pltpu.SemaphoreType.DMA,
            pltpu.SemaphoreType.DMA,
            pltpu.SemaphoreType.DMA,
        ],
    )
    def k(y_hbm, src_hbm, dst_hbm, z_hbm, out_hbm,
          sidx0, sidx1, didx, rows0, rows1, acc, g0, g1, s0, s1):
        c = lax.axis_index("c")
        s = lax.axis_index("s")
        wid = s * _NC + c
        row0 = pl.multiple_of(wid * _RPT, 8)
        sidx = [sidx0, sidx1]
        rows = [rows0, rows1]
        gs = [g0, g1]
        ss = [s0, s1]
        e0 = pl.multiple_of(wid * _RPT * C, 8)

        def load_sidx(j, b):
            base = pl.multiple_of(e0 + j * C, 8)
            pltpu.sync_copy(src_hbm.at[pl.ds(base, C)], sidx[b])

        pltpu.sync_copy(dst_hbm.at[pl.ds(row0, _RPT)], didx)

        @pl.when(s == 0)
        def _():
            pltpu.sync_copy(
                z_hbm.at[pl.ds(0, rpt8 + rem + _PAD)],
                acc.at[pl.ds(0, rpt8 + rem + _PAD)],
            )

        @pl.when(s > 0)
        def _():
            r0 = pl.multiple_of(rpt8 * s + rem + _PAD, 8)
            pltpu.sync_copy(z_hbm.at[pl.ds(r0, rpt8)], acc.at[pl.ds(r0, rpt8)])

        plsc.subcore_barrier()

        def step(j, b, o):
            @pl.when(j > 0)
            def _():
                pltpu.make_async_copy(
                    rows[o], acc.at[didx.at[j - 1]], ss[o]
                ).wait()

            @pl.when(j + 1 < _RPT)
            def _():
                load_sidx(j + 1, o)

            pltpu.make_async_copy(y_hbm.at[sidx[b]], rows[b], gs[b]).wait()

            @pl.when(j + 1 < _RPT)
            def _():
                pltpu.async_copy(y_hbm.at[sidx[o]], rows[o], gs[o])

            pltpu.async_copy(rows[b], acc.at[didx.at[j]], ss[b], add=True)

        load_sidx(0, 0)
        pltpu.async_copy(y_hbm.at[sidx0], rows0, g0)

        def body(i, carry):
            step(2 * i, 0, 1)
            step(2 * i + 1, 1, 0)
            return carry

        lax.fori_loop(0, _RPT // 2, body, 0)
        pltpu.make_async_copy(rows[1], acc.at[didx.at[_RPT - 1]], ss[1]).wait()

        plsc.subcore_barrier()

        @pl.when(s == 0)
        def _():
            pltpu.sync_copy(
                acc.at[pl.ds(0, rpt8 + rem)],
                out_hbm.at[c].at[pl.ds(0, rpt8 + rem)],
            )

        @pl.when(s > 0)
        def _():
            r0 = pl.multiple_of(rpt8 * s + rem, 8)
            pltpu.sync_copy(
                acc.at[pl.ds(r0, rpt8)], out_hbm.at[c].at[pl.ds(r0, rpt8)]
            )

    return k(y, srcflat, dst2, zeros_npd)


def _tc_first_body(x_ref, w_ref, p0_ref, p1_ref, y_ref, dinv_ref):
    deg = p0_ref[...] + p1_ref[...] + 1.0
    dinv = lax.rsqrt(deg)
    dinv_ref[...] = dinv
    xw = jnp.dot(x_ref[...], w_ref[...], preferred_element_type=jnp.float32)
    y_ref[...] = xw * dinv


def _tc_first(x, W, p0, p1):
    N, K = x.shape
    Dh = W.shape[1]
    B = 1000
    return pl.pallas_call(
        _tc_first_body,
        grid=(N // B,),
        in_specs=[
            pl.BlockSpec((B, K), lambda i: (i, 0)),
            pl.BlockSpec((K, Dh), lambda i: (0, 0)),
            pl.BlockSpec((B, 1), lambda i: (i, 0)),
            pl.BlockSpec((B, 1), lambda i: (i, 0)),
        ],
        out_specs=[
            pl.BlockSpec((B, Dh), lambda i: (i, 0)),
            pl.BlockSpec((B, 1), lambda i: (i, 0)),
        ],
        out_shape=[
            jax.ShapeDtypeStruct((N, Dh), jnp.float32),
            jax.ShapeDtypeStruct((N, 1), jnp.float32),
        ],
    )(x, W, p0, p1)


def _tc_l1_body(p0_ref, p1_ref, y_ref, dinv_ref, b_ref, u_ref):
    dinv = dinv_ref[...]
    h = dinv * (p0_ref[...] + p1_ref[...] + y_ref[...]) + b_ref[...]
    h = jnp.maximum(h, 0.0)
    u_ref[...] = h * dinv


def _tc_l1(p, y, dinv, b):
    N, D = y.shape
    B = 1000
    return pl.pallas_call(
        _tc_l1_body,
        grid=(N // B,),
        in_specs=[
            pl.BlockSpec((B, D), lambda i: (i, 0)),
            pl.BlockSpec((B, D), lambda i: (i, 0)),
            pl.BlockSpec((B, D), lambda i: (i, 0)),
            pl.BlockSpec((B, 1), lambda i: (i, 0)),
            pl.BlockSpec((1, D), lambda i: (0, 0)),
        ],
        out_specs=pl.BlockSpec((B, D), lambda i: (i, 0)),
        out_shape=jax.ShapeDtypeStruct((N, D), jnp.float32),
    )(p[0], p[1], y, dinv, b)


def _tc_l2_body(p0_ref, p1_ref, u_ref, dinv_ref, b2_ref, w2_ref, w3_ref, o_ref):
    dinv = dinv_ref[...]
    t = dinv * (p0_ref[...] + p1_ref[...] + u_ref[...])
    h2 = jnp.dot(t, w2_ref[...], preferred_element_type=jnp.float32) + b2_ref[...]
    y3 = jnp.dot(h2, w3_ref[...], preferred_element_type=jnp.float32) * dinv
    o_ref[...] = jnp.concatenate([y3, jnp.zeros_like(y3)], axis=1)


def _tc_l2(p, u, dinv, b2, W2, W3):
    N, D = u.shape
    B = 1000
    return pl.pallas_call(
        _tc_l2_body,
        grid=(N // B,),
        in_specs=[
            pl.BlockSpec((B, D), lambda i: (i, 0)),
            pl.BlockSpec((B, D), lambda i: (i, 0)),
            pl.BlockSpec((B, D), lambda i: (i, 0)),
            pl.BlockSpec((B, 1), lambda i: (i, 0)),
            pl.BlockSpec((1, W2.shape[1]), lambda i: (0, 0)),
            pl.BlockSpec(W2.shape, lambda i: (0, 0)),
            pl.BlockSpec(W3.shape, lambda i: (0, 0)),
        ],
        out_specs=pl.BlockSpec((B, D), lambda i: (i, 0)),
        out_shape=jax.ShapeDtypeStruct((N, D), jnp.float32),
    )(p[0], p[1], u, dinv, b2, W2, W3)


def _tc_l3_body(p0_ref, p1_ref, y_ref, dinv_ref, b_ref, o_ref, *, dout):
    v = dinv_ref[...] * (p0_ref[...] + p1_ref[...] + y_ref[...]) + b_ref[...]
    o_ref[...] = v[:, :dout]


def _tc_l3(p, y3p, dinv, b3, dout):
    N, D = y3p.shape
    B = 1000
    return pl.pallas_call(
        functools.partial(_tc_l3_body, dout=dout),
        grid=(N // B,),
        in_specs=[
            pl.BlockSpec((B, D), lambda i: (i, 0)),
            pl.BlockSpec((B, D), lambda i: (i, 0)),
            pl.BlockSpec((B, D), lambda i: (i, 0)),
            pl.BlockSpec((B, 1), lambda i: (i, 0)),
            pl.BlockSpec((1, D), lambda i: (0, 0)),
        ],
        out_specs=pl.BlockSpec((B, dout), lambda i: (i, 0)),
        out_shape=jax.ShapeDtypeStruct((N, dout), jnp.float32),
    )(p[0], p[1], y3p, dinv, b3)


def kernel(x, edge_index, W1, b1, W2, b2, W3, b3):
    N, _ = x.shape
    dout = W3.shape[1]
    src = edge_index[0]
    dst = edge_index[1]

    R = src.shape[0] // _C
    pad_rows = _NW * _RPT - R
    src2 = jnp.concatenate(
        [src.reshape(R, _C), jnp.zeros((pad_rows, _C), src.dtype)]
    )
    dst2 = jnp.concatenate(
        [dst.reshape(R, _C), jnp.full((pad_rows, _C), N, dst.dtype)]
    )
    srcflat = src2.reshape(-1)

    zeros_np = jnp.zeros((N + _PAD,), jnp.float32)
    degp0, degp1 = _sc_degree(dst2, zeros_np)
    degp0 = degp0[:N]
    degp1 = degp1[:N]

    y1, dinv = _tc_first(x, W1, degp0.reshape(N, 1), degp1.reshape(N, 1))

    z128 = jnp.zeros((N + _PAD, W1.shape[1]), jnp.float32)

    p1 = _sc_aggregate(y1, srcflat, dst2, z128)
    u = _tc_l1(p1, y1, dinv, b1.reshape(1, -1))
    p2 = _sc_aggregate(u, srcflat, dst2, z128)
    y3p = _tc_l2(p2, u, dinv, b2.reshape(1, -1), W2, W3)
    p3 = _sc_aggregate(y3p, srcflat, dst2, z128)
    b3p = jnp.concatenate([b3, jnp.zeros_like(b3)]).reshape(1, -1)
    out = _tc_l3(p3, y3p, dinv, b3p, dout)
    return out


# speedup vs baseline: 2.6301x; 2.6301x over previous
import functools

import jax
import jax.numpy as jnp
from jax import lax
from jax.experimental import pallas as pl
from jax.experimental.pallas import tpu as pltpu
from jax.experimental.pallas import tpu_sc as plsc

_NC = 2
_NS = 16
_NW = _NC * _NS
_C = 128


def _sc_degree(dst, zeros_n):
    (E,) = dst.shape
    (N,) = zeros_n.shape
    ept = E // _NW
    nfull = ept // _C
    tail = ept - nfull * _C

    mesh = plsc.VectorSubcoreMesh(core_axis_name="c", subcore_axis_name="s")

    @functools.partial(
        pl.kernel,
        out_type=(
            jax.ShapeDtypeStruct((N,), jnp.float32),
            jax.ShapeDtypeStruct((N,), jnp.float32),
        ),
        mesh=mesh,
        scratch_types=[
            pltpu.VMEM((_C,), jnp.int32),
            pltpu.VMEM((_C,), jnp.int32),
            pltpu.VMEM((tail,), jnp.int32),
            pltpu.VMEM((_C,), jnp.float32),
            pltpu.VMEM_SHARED((N,), jnp.float32),
            pltpu.SemaphoreType.DMA,
            pltpu.SemaphoreType.DMA,
        ],
    )
    def k(dst_hbm, z_hbm, out0_hbm, out1_hbm, didx0, didx1, tdidx, ones,
          acc, s0, s1):
        c = lax.axis_index("c")
        s = lax.axis_index("s")
        wid = s * _NC + c
        e0 = pl.multiple_of(wid * ept, 8)
        didx = [didx0, didx1]
        ss = [s0, s1]
        for i in range(_C // 16):
            ones[pl.ds(i * 16, 16)] = jnp.ones((16,), jnp.float32)

        @pl.when(s == 0)
        def _():
            pltpu.sync_copy(z_hbm, acc)

        plsc.subcore_barrier()

        def load_idx(j, b):
            base = pl.multiple_of(e0 + j * _C, 8)
            pltpu.sync_copy(dst_hbm.at[pl.ds(base, _C)], didx[b])

        def step(j, b, o):
            @pl.when(j > 0)
            def _():
                pltpu.make_async_copy(ones, acc.at[didx[o]], ss[o]).wait()

            @pl.when(j + 1 < nfull)
            def _():
                load_idx(j + 1, o)

            pltpu.async_copy(ones, acc.at[didx[b]], ss[b], add=True)

        load_idx(0, 0)

        def body(i, carry):
            step(2 * i, 0, 1)
            step(2 * i + 1, 1, 0)
            return carry

        lax.fori_loop(0, nfull // 2, body, 0)
        pltpu.make_async_copy(ones, acc.at[didx[1]], ss[1]).wait()

        if tail > 0:
            tbase = pl.multiple_of(e0 + nfull * _C, 8)
            pltpu.sync_copy(dst_hbm.at[pl.ds(tbase, tail)], tdidx)
            pltpu.sync_copy(ones.at[pl.ds(0, tail)], acc.at[tdidx], add=True)

        plsc.subcore_barrier()

        @pl.when(jnp.logical_and(s == 0, c == 0))
        def _():
            pltpu.sync_copy(acc, out0_hbm)

        @pl.when(jnp.logical_and(s == 0, c == 1))
        def _():
            pltpu.sync_copy(acc, out1_hbm)

    return k(dst, zeros_n)


def _sc_aggregate(y, src, dst, zeros_nd):
    N, D = y.shape
    (E,) = src.shape
    ept = E // _NW
    nfull = ept // _C
    tail = ept - nfull * _C
    assert nfull % 3 == 0
    rpt8 = (N // (_NS * 8)) * 8
    rem = N - _NS * rpt8

    mesh = plsc.VectorSubcoreMesh(core_axis_name="c", subcore_axis_name="s")

    @functools.partial(
        pl.kernel,
        out_type=jax.ShapeDtypeStruct((_NC, N, D), jnp.float32),
        mesh=mesh,
        scratch_types=[
            pltpu.VMEM((_C,), jnp.int32),
            pltpu.VMEM((_C,), jnp.int32),
            pltpu.VMEM((_C,), jnp.int32),
            pltpu.VMEM((_C,), jnp.int32),
            pltpu.VMEM((_C,), jnp.int32),
            pltpu.VMEM((_C,), jnp.int32),
            pltpu.VMEM((_C, D), jnp.float32),
            pltpu.VMEM((_C, D), jnp.float32),
            pltpu.VMEM((_C, D), jnp.float32),
            pltpu.VMEM((tail,), jnp.int32),
            pltpu.VMEM_SHARED((N, D), jnp.float32),
            pltpu.SemaphoreType.DMA,
            pltpu.SemaphoreType.DMA,
            pltpu.SemaphoreType.DMA,
            pltpu.SemaphoreType.DMA,
            pltpu.SemaphoreType.DMA,
            pltpu.SemaphoreType.DMA,
        ],
    )
    def k(y_hbm, src_hbm, dst_hbm, z_hbm, out_hbm,
          sidx0, sidx1, sidx2, didx0, didx1, didx2,
          rows0, rows1, rows2, tdidx, acc, g0, g1, g2, s0, s1, s2):
        c = lax.axis_index("c")
        s = lax.axis_index("s")
        wid = s * _NC + c
        e0 = pl.multiple_of(wid * ept, 8)
        sidx = [sidx0, sidx1, sidx2]
        didx = [didx0, didx1, didx2]
        rows = [rows0, rows1, rows2]
        gs = [g0, g1, g2]
        ss = [s0, s1, s2]

        @pl.when(s == 0)
        def _():
            pltpu.sync_copy(
                z_hbm.at[pl.ds(0, rpt8 + rem)], acc.at[pl.ds(0, rpt8 + rem)]
            )

        @pl.when(s > 0)
        def _():
            r0 = pl.multiple_of(rpt8 * s + rem, 8)
            pltpu.sync_copy(z_hbm.at[pl.ds(r0, rpt8)], acc.at[pl.ds(r0, rpt8)])

        plsc.subcore_barrier()

        def load_idx(j, b):
            base = pl.multiple_of(e0 + j * _C, 8)
            pltpu.sync_copy(src_hbm.at[pl.ds(base, _C)], sidx[b])
            pltpu.sync_copy(dst_hbm.at[pl.ds(base, _C)], didx[b])

        def step(j, b, n):
            @pl.when(j >= 2)
            def _():
                pltpu.make_async_copy(rows[n], acc.at[didx[n]], ss[n]).wait()

            @pl.when(j + 1 < nfull)
            def _():
                load_idx(j + 1, n)

            pltpu.make_async_copy(y_hbm.at[sidx[b]], rows[b], gs[b]).wait()

            @pl.when(j + 1 < nfull)
            def _():
                pltpu.async_copy(y_hbm.at[sidx[n]], rows[n], gs[n])

            pltpu.async_copy(rows[b], acc.at[didx[b]], ss[b], add=True)

        load_idx(0, 0)
        pltpu.async_copy(y_hbm.at[sidx0], rows0, g0)

        def body(i, carry):
            j = 3 * i
            step(j, 0, 1)
            step(j + 1, 1, 2)
            step(j + 2, 2, 0)
            return carry

        lax.fori_loop(0, nfull // 3, body, 0)
        pltpu.make_async_copy(rows[1], acc.at[didx[1]], ss[1]).wait()
        pltpu.make_async_copy(rows[2], acc.at[didx[2]], ss[2]).wait()

        if tail > 0:
            tbase = pl.multiple_of(e0 + nfull * _C, 8)
            pltpu.sync_copy(src_hbm.at[pl.ds(tbase, tail)], sidx0.at[pl.ds(0, tail)])
            pltpu.sync_copy(dst_hbm.at[pl.ds(tbase, tail)], tdidx)
            pltpu.async_copy(
                y_hbm.at[sidx0.at[pl.ds(0, tail)]], rows0.at[pl.ds(0, tail)], g0
            ).wait()
            pltpu.sync_copy(rows0.at[pl.ds(0, tail)], acc.at[tdidx], add=True)

        plsc.subcore_barrier()

        @pl.when(s == 0)
        def _():
            pltpu.sync_copy(
                acc.at[pl.ds(0, rpt8 + rem)],
                out_hbm.at[c].at[pl.ds(0, rpt8 + rem)],
            )

        @pl.when(s > 0)
        def _():
            r0 = pl.multiple_of(rpt8 * s + rem, 8)
            pltpu.sync_copy(
                acc.at[pl.ds(r0, rpt8)], out_hbm.at[c].at[pl.ds(r0, rpt8)]
            )

    return k(y, src, dst, zeros_nd)


def _tc_first_body(x_ref, w_ref, p0_ref, p1_ref, y_ref, dinv_ref):
    deg = p0_ref[...] + p1_ref[...] + 1.0
    dinv = lax.rsqrt(deg)
    dinv_ref[...] = dinv
    xw = jnp.dot(x_ref[...], w_ref[...], preferred_element_type=jnp.float32)
    y_ref[...] = xw * dinv


def _tc_first(x, W, p0, p1):
    N, K = x.shape
    Dh = W.shape[1]
    B = 1000
    return pl.pallas_call(
        _tc_first_body,
        grid=(N // B,),
        in_specs=[
            pl.BlockSpec((B, K), lambda i: (i, 0)),
            pl.BlockSpec((K, Dh), lambda i: (0, 0)),
            pl.BlockSpec((B, 1), lambda i: (i, 0)),
            pl.BlockSpec((B, 1), lambda i: (i, 0)),
        ],
        out_specs=[
            pl.BlockSpec((B, Dh), lambda i: (i, 0)),
            pl.BlockSpec((B, 1), lambda i: (i, 0)),
        ],
        out_shape=[
            jax.ShapeDtypeStruct((N, Dh), jnp.float32),
            jax.ShapeDtypeStruct((N, 1), jnp.float32),
        ],
    )(x, W, p0, p1)


def _tc_l1_body(p0_ref, p1_ref, y_ref, dinv_ref, b_ref, u_ref):
    dinv = dinv_ref[...]
    h = dinv * (p0_ref[...] + p1_ref[...] + y_ref[...]) + b_ref[...]
    h = jnp.maximum(h, 0.0)
    u_ref[...] = h * dinv


def _tc_l1(p, y, dinv, b):
    N, D = y.shape
    B = 1000
    return pl.pallas_call(
        _tc_l1_body,
        grid=(N // B,),
        in_specs=[
            pl.BlockSpec((B, D), lambda i: (i, 0)),
            pl.BlockSpec((B, D), lambda i: (i, 0)),
            pl.BlockSpec((B, D), lambda i: (i, 0)),
            pl.BlockSpec((B, 1), lambda i: (i, 0)),
            pl.BlockSpec((1, D), lambda i: (0, 0)),
        ],
        out_specs=pl.BlockSpec((B, D), lambda i: (i, 0)),
        out_shape=jax.ShapeDtypeStruct((N, D), jnp.float32),
    )(p[0], p[1], y, dinv, b)


def _tc_l2_body(p0_ref, p1_ref, u_ref, dinv_ref, b2_ref, w2_ref, w3_ref, o_ref):
    dinv = dinv_ref[...]
    t = dinv * (p0_ref[...] + p1_ref[...] + u_ref[...])
    h2 = jnp.dot(t, w2_ref[...], preferred_element_type=jnp.float32) + b2_ref[...]
    y3 = jnp.dot(h2, w3_ref[...], preferred_element_type=jnp.float32) * dinv
    o_ref[...] = jnp.concatenate([y3, jnp.zeros_like(y3)], axis=1)


def _tc_l2(p, u, dinv, b2, W2, W3):
    N, D = u.shape
    B = 1000
    return pl.pallas_call(
        _tc_l2_body,
        grid=(N // B,),
        in_specs=[
            pl.BlockSpec((B, D), lambda i: (i, 0)),
            pl.BlockSpec((B, D), lambda i: (i, 0)),
            pl.BlockSpec((B, D), lambda i: (i, 0)),
            pl.BlockSpec((B, 1), lambda i: (i, 0)),
            pl.BlockSpec((1, W2.shape[1]), lambda i: (0, 0)),
            pl.BlockSpec(W2.shape, lambda i: (0, 0)),
            pl.BlockSpec(W3.shape, lambda i: (0, 0)),
        ],
        out_specs=pl.BlockSpec((B, D), lambda i: (i, 0)),
        out_shape=jax.ShapeDtypeStruct((N, D), jnp.float32),
    )(p[0], p[1], u, dinv, b2, W2, W3)


def _tc_l3_body(p0_ref, p1_ref, y_ref, dinv_ref, b_ref, o_ref, *, dout):
    v = dinv_ref[...] * (p0_ref[...] + p1_ref[...] + y_ref[...]) + b_ref[...]
    o_ref[...] = v[:, :dout]


def _tc_l3(p, y3p, dinv, b3, dout):
    N, D = y3p.shape
    B = 1000
    return pl.pallas_call(
        functools.partial(_tc_l3_body, dout=dout),
        grid=(N // B,),
        in_specs=[
            pl.BlockSpec((B, D), lambda i: (i, 0)),
            pl.BlockSpec((B, D), lambda i: (i, 0)),
            pl.BlockSpec((B, D), lambda i: (i, 0)),
            pl.BlockSpec((B, 1), lambda i: (i, 0)),
            pl.BlockSpec((1, D), lambda i: (0, 0)),
        ],
        out_specs=pl.BlockSpec((B, dout), lambda i: (i, 0)),
        out_shape=jax.ShapeDtypeStruct((N, dout), jnp.float32),
    )(p[0], p[1], y3p, dinv, b3)


def kernel(x, edge_index, W1, b1, W2, b2, W3, b3):
    N, _ = x.shape
    dout = W3.shape[1]
    src = edge_index[0]
    dst = edge_index[1]

    zeros_n = jnp.zeros((N,), jnp.float32)
    degp0, degp1 = _sc_degree(dst, zeros_n)

    y1, dinv = _tc_first(x, W1, degp0.reshape(N, 1), degp1.reshape(N, 1))

    z128 = jnp.zeros((N, W1.shape[1]), jnp.float32)

    p1 = _sc_aggregate(y1, src, dst, z128)
    u = _tc_l1(p1, y1, dinv, b1.reshape(1, -1))
    p2 = _sc_aggregate(u, src, dst, z128)
    y3p = _tc_l2(p2, u, dinv, b2.reshape(1, -1), W2, W3)
    p3 = _sc_aggregate(y3p, src, dst, z128)
    b3p = jnp.concatenate([b3, jnp.zeros_like(b3)]).reshape(1, -1)
    out = _tc_l3(p3, y3p, dinv, b3p, dout)
    return out


# speedup vs baseline: 2.6596x; 1.0112x over previous
import functools

import jax
import jax.numpy as jnp
from jax import lax
from jax.experimental import pallas as pl
from jax.experimental.pallas import tpu as pltpu
from jax.experimental.pallas import tpu_sc as plsc

_NC = 2
_NS = 16
_NW = _NC * _NS
_C = 128


def _sc_degree(dst, zeros_n):
    (E,) = dst.shape
    (N,) = zeros_n.shape
    ept = E // _NW
    nfull = ept // _C
    tail = ept - nfull * _C
    assert nfull % 3 == 0

    mesh = plsc.VectorSubcoreMesh(core_axis_name="c", subcore_axis_name="s")

    @functools.partial(
        pl.kernel,
        out_type=(
            jax.ShapeDtypeStruct((N,), jnp.float32),
            jax.ShapeDtypeStruct((N,), jnp.float32),
        ),
        mesh=mesh,
        scratch_types=[
            pltpu.VMEM((_C,), jnp.int32),
            pltpu.VMEM((_C,), jnp.int32),
            pltpu.VMEM((_C,), jnp.int32),
            pltpu.VMEM((tail,), jnp.int32),
            pltpu.VMEM((_C,), jnp.float32),
            pltpu.VMEM_SHARED((N,), jnp.float32),
            pltpu.SemaphoreType.DMA,
            pltpu.SemaphoreType.DMA,
            pltpu.SemaphoreType.DMA,
        ],
    )
    def k(dst_hbm, z_hbm, out0_hbm, out1_hbm, didx0, didx1, didx2, tdidx,
          ones, acc, s0, s1, s2):
        c = lax.axis_index("c")
        s = lax.axis_index("s")
        wid = s * _NC + c
        e0 = pl.multiple_of(wid * ept, 8)
        didx = [didx0, didx1, didx2]
        ss = [s0, s1, s2]
        for i in range(_C // 16):
            ones[pl.ds(i * 16, 16)] = jnp.ones((16,), jnp.float32)

        @pl.when(s == 0)
        def _():
            pltpu.sync_copy(z_hbm, acc)

        plsc.subcore_barrier()

        def load_idx(j, b):
            base = pl.multiple_of(e0 + j * _C, 8)
            pltpu.sync_copy(dst_hbm.at[pl.ds(base, _C)], didx[b])

        def step(j, b, n):
            @pl.when(j >= 2)
            def _():
                pltpu.make_async_copy(ones, acc.at[didx[n]], ss[n]).wait()

            @pl.when(j + 1 < nfull)
            def _():
                load_idx(j + 1, n)

            pltpu.async_copy(ones, acc.at[didx[b]], ss[b], add=True)

        load_idx(0, 0)

        def body(i, carry):
            j = 3 * i
            step(j, 0, 1)
            step(j + 1, 1, 2)
            step(j + 2, 2, 0)
            return carry

        lax.fori_loop(0, nfull // 3, body, 0)
        pltpu.make_async_copy(ones, acc.at[didx[1]], ss[1]).wait()
        pltpu.make_async_copy(ones, acc.at[didx[2]], ss[2]).wait()

        if tail > 0:
            tbase = pl.multiple_of(e0 + nfull * _C, 8)
            pltpu.sync_copy(dst_hbm.at[pl.ds(tbase, tail)], tdidx)
            pltpu.sync_copy(ones.at[pl.ds(0, tail)], acc.at[tdidx], add=True)

        plsc.subcore_barrier()

        @pl.when(jnp.logical_and(s == 0, c == 0))
        def _():
            pltpu.sync_copy(acc, out0_hbm)

        @pl.when(jnp.logical_and(s == 0, c == 1))
        def _():
            pltpu.sync_copy(acc, out1_hbm)

    return k(dst, zeros_n)


def _sc_aggregate(y, src, dst, zeros_nd):
    N, D = y.shape
    (E,) = src.shape
    ept = E // _NW
    nfull = ept // _C
    tail = ept - nfull * _C
    assert nfull % 3 == 0
    rpt8 = (N // (_NS * 8)) * 8
    rem = N - _NS * rpt8

    mesh = plsc.VectorSubcoreMesh(core_axis_name="c", subcore_axis_name="s")

    @functools.partial(
        pl.kernel,
        out_type=jax.ShapeDtypeStruct((_NC, N, D), jnp.float32),
        mesh=mesh,
        scratch_types=[
            pltpu.VMEM((_C,), jnp.int32),
            pltpu.VMEM((_C,), jnp.int32),
            pltpu.VMEM((_C,), jnp.int32),
            pltpu.VMEM((_C,), jnp.int32),
            pltpu.VMEM((_C,), jnp.int32),
            pltpu.VMEM((_C,), jnp.int32),
            pltpu.VMEM((_C, D), jnp.float32),
            pltpu.VMEM((_C, D), jnp.float32),
            pltpu.VMEM((_C, D), jnp.float32),
            pltpu.VMEM((tail,), jnp.int32),
            pltpu.VMEM_SHARED((N, D), jnp.float32),
            pltpu.SemaphoreType.DMA,
            pltpu.SemaphoreType.DMA,
            pltpu.SemaphoreType.DMA,
            pltpu.SemaphoreType.DMA,
            pltpu.SemaphoreType.DMA,
            pltpu.SemaphoreType.DMA,
        ],
    )
    def k(y_hbm, src_hbm, dst_hbm, z_hbm, out_hbm,
          sidx0, sidx1, sidx2, didx0, didx1, didx2,
          rows0, rows1, rows2, tdidx, acc, g0, g1, g2, s0, s1, s2):
        c = lax.axis_index("c")
        s = lax.axis_index("s")
        wid = s * _NC + c
        e0 = pl.multiple_of(wid * ept, 8)
        sidx = [sidx0, sidx1, sidx2]
        didx = [didx0, didx1, didx2]
        rows = [rows0, rows1, rows2]
        gs = [g0, g1, g2]
        ss = [s0, s1, s2]

        @pl.when(s == 0)
        def _():
            pltpu.sync_copy(
                z_hbm.at[pl.ds(0, rpt8 + rem)], acc.at[pl.ds(0, rpt8 + rem)]
            )

        @pl.when(s > 0)
        def _():
            r0 = pl.multiple_of(rpt8 * s + rem, 8)
            pltpu.sync_copy(z_hbm.at[pl.ds(r0, rpt8)], acc.at[pl.ds(r0, rpt8)])

        plsc.subcore_barrier()

        def load_idx(j, b):
            base = pl.multiple_of(e0 + j * _C, 8)
            pltpu.sync_copy(src_hbm.at[pl.ds(base, _C)], sidx[b])
            pltpu.sync_copy(dst_hbm.at[pl.ds(base, _C)], didx[b])

        def step(j, b, n):
            @pl.when(j >= 2)
            def _():
                pltpu.make_async_copy(rows[n], acc.at[didx[n]], ss[n]).wait()

            @pl.when(j + 1 < nfull)
            def _():
                load_idx(j + 1, n)

            pltpu.make_async_copy(y_hbm.at[sidx[b]], rows[b], gs[b]).wait()

            @pl.when(j + 1 < nfull)
            def _():
                pltpu.async_copy(y_hbm.at[sidx[n]], rows[n], gs[n])

            pltpu.async_copy(rows[b], acc.at[didx[b]], ss[b], add=True)

        load_idx(0, 0)
        pltpu.async_copy(y_hbm.at[sidx0], rows0, g0)

        def body(i, carry):
            j = 3 * i
            step(j, 0, 1)
            step(j + 1, 1, 2)
            step(j + 2, 2, 0)
            return carry

        lax.fori_loop(0, nfull // 3, body, 0)
        pltpu.make_async_copy(rows[1], acc.at[didx[1]], ss[1]).wait()
        pltpu.make_async_copy(rows[2], acc.at[didx[2]], ss[2]).wait()

        if tail > 0:
            tbase = pl.multiple_of(e0 + nfull * _C, 8)
            pltpu.sync_copy(src_hbm.at[pl.ds(tbase, tail)], sidx0.at[pl.ds(0, tail)])
            pltpu.sync_copy(dst_hbm.at[pl.ds(tbase, tail)], tdidx)
            pltpu.async_copy(
                y_hbm.at[sidx0.at[pl.ds(0, tail)]], rows0.at[pl.ds(0, tail)], g0
            ).wait()
            pltpu.sync_copy(rows0.at[pl.ds(0, tail)], acc.at[tdidx], add=True)

        plsc.subcore_barrier()

        @pl.when(s == 0)
        def _():
            pltpu.sync_copy(
                acc.at[pl.ds(0, rpt8 + rem)],
                out_hbm.at[c].at[pl.ds(0, rpt8 + rem)],
            )

        @pl.when(s > 0)
        def _():
            r0 = pl.multiple_of(rpt8 * s + rem, 8)
            pltpu.sync_copy(
                acc.at[pl.ds(r0, rpt8)], out_hbm.at[c].at[pl.ds(r0, rpt8)]
            )

    return k(y, src, dst, zeros_nd)


def _tc_first_body(x_ref, w_ref, p0_ref, p1_ref, y_ref, dinv_ref):
    deg = p0_ref[...] + p1_ref[...] + 1.0
    dinv = lax.rsqrt(deg)
    dinv_ref[...] = dinv
    xw = jnp.dot(x_ref[...], w_ref[...], preferred_element_type=jnp.float32)
    y_ref[...] = xw * dinv


def _tc_first(x, W, p0, p1):
    N, K = x.shape
    Dh = W.shape[1]
    B = 1000
    return pl.pallas_call(
        _tc_first_body,
        grid=(N // B,),
        in_specs=[
            pl.BlockSpec((B, K), lambda i: (i, 0)),
            pl.BlockSpec((K, Dh), lambda i: (0, 0)),
            pl.BlockSpec((B, 1), lambda i: (i, 0)),
            pl.BlockSpec((B, 1), lambda i: (i, 0)),
        ],
        out_specs=[
            pl.BlockSpec((B, Dh), lambda i: (i, 0)),
            pl.BlockSpec((B, 1), lambda i: (i, 0)),
        ],
        out_shape=[
            jax.ShapeDtypeStruct((N, Dh), jnp.float32),
            jax.ShapeDtypeStruct((N, 1), jnp.float32),
        ],
    )(x, W, p0, p1)


def _tc_l1_body(p0_ref, p1_ref, y_ref, dinv_ref, b_ref, u_ref):
    dinv = dinv_ref[...]
    h = dinv * (p0_ref[...] + p1_ref[...] + y_ref[...]) + b_ref[...]
    h = jnp.maximum(h, 0.0)
    u_ref[...] = h * dinv


def _tc_l1(p, y, dinv, b):
    N, D = y.shape
    B = 1000
    return pl.pallas_call(
        _tc_l1_body,
        grid=(N // B,),
        in_specs=[
            pl.BlockSpec((B, D), lambda i: (i, 0)),
            pl.BlockSpec((B, D), lambda i: (i, 0)),
            pl.BlockSpec((B, D), lambda i: (i, 0)),
            pl.BlockSpec((B, 1), lambda i: (i, 0)),
            pl.BlockSpec((1, D), lambda i: (0, 0)),
        ],
        out_specs=pl.BlockSpec((B, D), lambda i: (i, 0)),
        out_shape=jax.ShapeDtypeStruct((N, D), jnp.float32),
    )(p[0], p[1], y, dinv, b)


def _tc_l2_body(p0_ref, p1_ref, u_ref, dinv_ref, b2_ref, w2_ref, w3_ref, o_ref):
    dinv = dinv_ref[...]
    t = dinv * (p0_ref[...] + p1_ref[...] + u_ref[...])
    h2 = jnp.dot(t, w2_ref[...], preferred_element_type=jnp.float32) + b2_ref[...]
    y3 = jnp.dot(h2, w3_ref[...], preferred_element_type=jnp.float32) * dinv
    o_ref[...] = jnp.concatenate([y3, jnp.zeros_like(y3)], axis=1)


def _tc_l2(p, u, dinv, b2, W2, W3):
    N, D = u.shape
    B = 1000
    return pl.pallas_call(
        _tc_l2_body,
        grid=(N // B,),
        in_specs=[
            pl.BlockSpec((B, D), lambda i: (i, 0)),
            pl.BlockSpec((B, D), lambda i: (i, 0)),
            pl.BlockSpec((B, D), lambda i: (i, 0)),
            pl.BlockSpec((B, 1), lambda i: (i, 0)),
            pl.BlockSpec((1, W2.shape[1]), lambda i: (0, 0)),
            pl.BlockSpec(W2.shape, lambda i: (0, 0)),
            pl.BlockSpec(W3.shape, lambda i: (0, 0)),
        ],
        out_specs=pl.BlockSpec((B, D), lambda i: (i, 0)),
        out_shape=jax.ShapeDtypeStruct((N, D), jnp.float32),
    )(p[0], p[1], u, dinv, b2, W2, W3)


def _tc_l3_body(p0_ref, p1_ref, y_ref, dinv_ref, b_ref, o_ref, *, dout):
    v = dinv_ref[...] * (p0_ref[...] + p1_ref[...] + y_ref[...]) + b_ref[...]
    o_ref[...] = v[:, :dout]


def _tc_l3(p, y3p, dinv, b3, dout):
    N, D = y3p.shape
    B = 1000
    return pl.pallas_call(
        functools.partial(_tc_l3_body, dout=dout),
        grid=(N // B,),
        in_specs=[
            pl.BlockSpec((B, D), lambda i: (i, 0)),
            pl.BlockSpec((B, D), lambda i: (i, 0)),
            pl.BlockSpec((B, D), lambda i: (i, 0)),
            pl.BlockSpec((B, 1), lambda i: (i, 0)),
            pl.BlockSpec((1, D), lambda i: (0, 0)),
        ],
        out_specs=pl.BlockSpec((B, dout), lambda i: (i, 0)),
        out_shape=jax.ShapeDtypeStruct((N, dout), jnp.float32),
    )(p[0], p[1], y3p, dinv, b3)


def kernel(x, edge_index, W1, b1, W2, b2, W3, b3):
    N, _ = x.shape
    dout = W3.shape[1]
    src = edge_index[0]
    dst = edge_index[1]

    zeros_n = jnp.zeros((N,), jnp.float32)
    degp0, degp1 = _sc_degree(dst, zeros_n)

    y1, dinv = _tc_first(x, W1, degp0.reshape(N, 1), degp1.reshape(N, 1))

    z128 = jnp.zeros((N, W1.shape[1]), jnp.float32)

    p1 = _sc_aggregate(y1, src, dst, z128)
    u = _tc_l1(p1, y1, dinv, b1.reshape(1, -1))
    p2 = _sc_aggregate(u, src, dst, z128)
    y3p = _tc_l2(p2, u, dinv, b2.reshape(1, -1), W2, W3)
    p3 = _sc_aggregate(y3p, src, dst, z128)
    b3p = jnp.concatenate([b3, jnp.zeros_like(b3)]).reshape(1, -1)
    out = _tc_l3(p3, y3p, dinv, b3p, dout)
    return out
